# baseline (speedup 1.0000x reference)
import functools

import jax
import jax.numpy as jnp
from jax import lax
from jax.experimental import pallas as pl
from jax.experimental.pallas import tpu as pltpu
from jax.experimental.pallas import tpu_sc as plsc

_NW = 32
_SUB = 8


def _make_kernel(B, D, b_per_w):
    mesh = plsc.VectorSubcoreMesh(core_axis_name="c", subcore_axis_name="s")
    n_tiles_w = b_per_w // _SUB

    @functools.partial(
        pl.kernel,
        mesh=mesh,
        out_type=(
            jax.ShapeDtypeStruct((B // _SUB, _SUB, D), jnp.float32),
            jax.ShapeDtypeStruct((B // _SUB, _SUB, D), jnp.float32),
        ),
        scratch_types=[
            pltpu.VMEM((b_per_w,), jnp.int32),
            pltpu.VMEM((b_per_w,), jnp.int32),
            pltpu.SemaphoreType.DMA,
            pltpu.SemaphoreType.DMA,
        ],
    )
    def k(e1_hbm, q_hbm, e3_hbm, r3_hbm, out_h_hbm, out_q_hbm,
          eidx_v, qidx_v, sem_e, sem_q):
        wid = lax.axis_index("s") * 2 + lax.axis_index("c")
        base = wid * b_per_w
        tbase = wid * n_tiles_w

        pltpu.sync_copy(e1_hbm.at[pl.ds(base, b_per_w)], eidx_v)
        pltpu.sync_copy(q_hbm.at[pl.ds(base, b_per_w)], qidx_v)

        def row(i):
            return (lax.shift_right_logical(i, 3), lax.bitwise_and(i, _SUB - 1))

        def fire(g, carry):
            ve = eidx_v[pl.ds(g * 16, 16)]
            vq = qidx_v[pl.ds(g * 16, 16)]
            copies = []
            for k_ in range(16):
                j = g * 16 + k_
                dt = tbase + lax.shift_right_logical(j, 3)
                dr = lax.bitwise_and(j, _SUB - 1)
                te, re = row(ve[k_])
                tq, rq = row(vq[k_])
                copies.append(
                    pltpu.async_copy(e3_hbm.at[te, re], out_h_hbm.at[dt, dr],
                                     sem_e))
                copies.append(
                    pltpu.async_copy(r3_hbm.at[tq, rq], out_q_hbm.at[dt, dr],
                                     sem_q))
            for cp in copies:
                cp.wait()
            return carry

        lax.fori_loop(0, b_per_w // 16, fire, 0)

    return k


@jax.jit
def _gather2(batch_e1, batch_q, emb_e, emb_r):
    B = batch_e1.shape[0]
    D = emb_e.shape[1]
    b_per_w = B // _NW
    e3 = emb_e.reshape(-1, _SUB, D)
    r3 = emb_r.reshape(-1, _SUB, D)
    k = _make_kernel(B, D, b_per_w)
    out_h, out_q = k(batch_e1, batch_q, e3, r3)
    return out_h.reshape(B, D), out_q.reshape(B, D)


def kernel(batch_e1, batch_q, emb_e, emb_r):
    return _gather2(batch_e1.astype(jnp.int32), batch_q.astype(jnp.int32),
                    emb_e, emb_r)


# speedup vs baseline: 1.1586x; 1.1586x over previous
import functools

import jax
import jax.numpy as jnp
from jax import lax
from jax.experimental import pallas as pl
from jax.experimental.pallas import tpu as pltpu
from jax.experimental.pallas import tpu_sc as plsc

_NUM_WORKERS = 32
_CHUNK = 128


def _gather2(batch_e1, batch_q, emb_e, emb_r):
    B = batch_e1.shape[0]
    D = emb_e.shape[1]
    b_per_w = B // _NUM_WORKERS
    n_chunks = b_per_w // _CHUNK

    mesh = plsc.VectorSubcoreMesh(core_axis_name="c", subcore_axis_name="s")

    @functools.partial(
        pl.kernel,
        mesh=mesh,
        out_type=(
            jax.ShapeDtypeStruct((B, D), jnp.float32),
            jax.ShapeDtypeStruct((B, D), jnp.float32),
        ),
        scratch_types=[
            pltpu.VMEM((b_per_w,), jnp.int32),
            pltpu.VMEM((b_per_w,), jnp.int32),
            pltpu.VMEM((b_per_w, D), jnp.float32),
            pltpu.VMEM((b_per_w, D), jnp.float32),
            pltpu.SemaphoreType.DMA,
            pltpu.SemaphoreType.DMA,
        ],
        compiler_params=pltpu.CompilerParams(use_tc_tiling_on_sc=False),
    )
    def k(e1_hbm, q_hbm, emb_e_hbm, emb_r_hbm, out_h_hbm, out_q_hbm,
          idx_e, idx_q, rows_e, rows_q, sem_e, sem_q):
        wid = lax.axis_index("s") * 2 + lax.axis_index("c")
        base = wid * b_per_w
        pltpu.sync_copy(e1_hbm.at[pl.ds(base, b_per_w)], idx_e)
        pltpu.sync_copy(q_hbm.at[pl.ds(base, b_per_w)], idx_q)
        copies = []
        for j in range(n_chunks):
            s = pl.ds(j * _CHUNK, _CHUNK)
            copies.append(
                pltpu.async_copy(emb_e_hbm.at[idx_e.at[s]], rows_e.at[s], sem_e))
            copies.append(
                pltpu.async_copy(emb_r_hbm.at[idx_q.at[s]], rows_q.at[s], sem_q))
        for cp in copies:
            cp.wait()
        pltpu.sync_copy(rows_e, out_h_hbm.at[pl.ds(base, b_per_w)])
        pltpu.sync_copy(rows_q, out_q_hbm.at[pl.ds(base, b_per_w)])

    return k(batch_e1, batch_q, emb_e, emb_r)


def kernel(batch_e1, batch_q, emb_e, emb_r):
    if batch_e1.dtype != jnp.int32:
        batch_e1 = batch_e1.astype(jnp.int32)
        batch_q = batch_q.astype(jnp.int32)
    return _gather2(batch_e1, batch_q, emb_e, emb_r)
